# Initial kernel scaffold; baseline (speedup 1.0000x reference)
import functools

import jax
import jax.numpy as jnp
from jax import lax
from jax.experimental import pallas as pl
from jax.experimental.pallas import tpu as pltpu
from jax.experimental.pallas import tpu_sc as plsc

NC = 2
NS = 16
NW = NC * NS

N = 10000
E = 320000
D = 128

T = E // NW
K = 100
R = T // K
ZR = 125
ROWS_PER_TILE = N // NS


def _sc_agg_body(src_hbm, dst_hbm, x_hbm, out_hbm,
                 src_v, dst_v, rows_v, zbuf, acc, sem):
    c = lax.axis_index("c")
    s = lax.axis_index("s")
    wid = s * NC + c

    pltpu.sync_copy(src_hbm.at[wid], src_v)
    pltpu.sync_copy(dst_hbm.at[wid], dst_v)

    zeros16 = jnp.zeros((16,), jnp.float32)

    def zrow(i, carry):
        for cc in range(D // 16):
            zbuf[i, pl.ds(cc * 16, 16)] = zeros16
        return carry

    lax.fori_loop(0, ZR, zrow, 0)

    def zcp(i, carry):
        pltpu.sync_copy(zbuf, acc.at[pl.ds(s * ROWS_PER_TILE + i * ZR, ZR)])
        return carry

    lax.fori_loop(0, ROWS_PER_TILE // ZR, zcp, 0)
    plsc.subcore_barrier()

    def body(j, carry):
        pltpu.async_copy(x_hbm.at[src_v.at[j]], rows_v, sem).wait()
        pltpu.sync_copy(rows_v, acc.at[dst_v.at[j]], add=True)
        return carry

    lax.fori_loop(0, R, body, 0)
    plsc.subcore_barrier()

    pltpu.sync_copy(
        acc.at[pl.ds(s * ROWS_PER_TILE, ROWS_PER_TILE)],
        out_hbm.at[c, pl.ds(s * ROWS_PER_TILE, ROWS_PER_TILE)],
    )


_sc_agg = functools.partial(
    pl.kernel,
    out_type=jax.ShapeDtypeStruct((NC, N, D), jnp.float32),
    mesh=plsc.VectorSubcoreMesh(core_axis_name="c", subcore_axis_name="s"),
    scratch_types=[
        pltpu.VMEM((R, K), jnp.int32),
        pltpu.VMEM((R, K), jnp.int32),
        pltpu.VMEM((K, D), jnp.float32),
        pltpu.VMEM((ZR, D), jnp.float32),
        pltpu.VMEM_SHARED((N, D), jnp.float32),
        pltpu.SemaphoreType.DMA,
    ],
)(_sc_agg_body)


BN = 1000


def _mlp_body(p_ref, x_ref, sc_ref, w1_ref, b1_ref, w2_ref, b2_ref, o_ref):
    y = p_ref[0] + p_ref[1] + sc_ref[0, 0] * x_ref[...]
    h = jnp.dot(y, w1_ref[...], preferred_element_type=jnp.float32)
    h = jnp.maximum(h + b1_ref[...], 0.0)
    o = jnp.dot(h, w2_ref[...], preferred_element_type=jnp.float32)
    o_ref[...] = o + b2_ref[...]


def _mlp(partials, x, scale, W1, b1, W2, b2):
    grid = (N // BN,)
    return pl.pallas_call(
        _mlp_body,
        grid=grid,
        in_specs=[
            pl.BlockSpec((NC, BN, D), lambda i: (0, i, 0)),
            pl.BlockSpec((BN, D), lambda i: (i, 0)),
            pl.BlockSpec((1, 1), lambda i: (0, 0)),
            pl.BlockSpec((D, D), lambda i: (0, 0)),
            pl.BlockSpec((1, D), lambda i: (0, 0)),
            pl.BlockSpec((D, D), lambda i: (0, 0)),
            pl.BlockSpec((1, D), lambda i: (0, 0)),
        ],
        out_specs=pl.BlockSpec((BN, D), lambda i: (i, 0)),
        out_shape=jax.ShapeDtypeStruct((N, D), jnp.float32),
    )(partials, x, scale, W1, b1, W2, b2)


@jax.jit
def kernel(x, edge_index, W1, b1, W2, b2, eps):
    src = edge_index[0].reshape(NW, R, K)
    dst = edge_index[1].reshape(NW, R, K)
    partials = _sc_agg(src, dst, x)
    scale = (1.0 + eps[0]).reshape(1, 1)
    return _mlp(partials, x, scale, W1, b1.reshape(1, D), W2, b2.reshape(1, D))


# speedup vs baseline: 8.2874x; 8.2874x over previous
import functools

import jax
import jax.numpy as jnp
from jax import lax
from jax.experimental import pallas as pl
from jax.experimental.pallas import tpu as pltpu
from jax.experimental.pallas import tpu_sc as plsc

NC = 2
NS = 16
NW = NC * NS

N = 10000
E = 320000
D = 128

T = E // NW
K = 100
R = T // K
NP = 10240
ROWS_PER_TILE = NP // NS
ZR = 64
assert ROWS_PER_TILE % ZR == 0 and T == R * K and NP % NS == 0 and ZR <= K


def _sc_agg_body(src_hbm, dst_hbm, x_hbm, out_hbm,
                 src_v, dst_v, rows_v, acc, sem):
    c = lax.axis_index("c")
    s = lax.axis_index("s")
    wid = s * NC + c

    pltpu.sync_copy(src_hbm.at[wid], src_v)
    pltpu.sync_copy(dst_hbm.at[wid], dst_v)

    zeros16 = jnp.zeros((16,), jnp.float32)

    def zrow(i, carry):
        for cc in range(D // 16):
            rows_v[i, pl.ds(cc * 16, 16)] = zeros16
        return carry

    lax.fori_loop(0, ZR, zrow, 0)

    def zcp(i, carry):
        pltpu.sync_copy(rows_v.at[pl.ds(0, ZR)],
                        acc.at[pl.ds(s * ROWS_PER_TILE + i * ZR, ZR)])
        return carry

    lax.fori_loop(0, ROWS_PER_TILE // ZR, zcp, 0)
    plsc.subcore_barrier()

    def body(j, carry):
        pltpu.async_copy(x_hbm.at[src_v.at[j]], rows_v, sem).wait()
        pltpu.sync_copy(rows_v, acc.at[dst_v.at[j]], add=True)
        return carry

    lax.fori_loop(0, R, body, 0)
    plsc.subcore_barrier()

    pltpu.sync_copy(
        acc.at[pl.ds(s * ROWS_PER_TILE, ROWS_PER_TILE)],
        out_hbm.at[c, pl.ds(s * ROWS_PER_TILE, ROWS_PER_TILE)],
    )


_sc_agg = functools.partial(
    pl.kernel,
    out_type=jax.ShapeDtypeStruct((NC, NP, D), jnp.float32),
    mesh=plsc.VectorSubcoreMesh(core_axis_name="c", subcore_axis_name="s"),
    scratch_types=[
        pltpu.VMEM((R, K), jnp.int32),
        pltpu.VMEM((R, K), jnp.int32),
        pltpu.VMEM((K, D), jnp.float32),
        pltpu.VMEM_SHARED((NP, D), jnp.float32),
        pltpu.SemaphoreType.DMA,
    ],
)(_sc_agg_body)


BN = 1000


def _mlp_body(p_ref, x_ref, sc_ref, w1_ref, b1_ref, w2_ref, b2_ref, o_ref):
    y = p_ref[0] + p_ref[1] + sc_ref[0, 0] * x_ref[...]
    h = jnp.dot(y, w1_ref[...], preferred_element_type=jnp.float32)
    h = jnp.maximum(h + b1_ref[...], 0.0)
    o = jnp.dot(h, w2_ref[...], preferred_element_type=jnp.float32)
    o_ref[...] = o + b2_ref[...]


def _mlp(partials, x, scale, W1, b1, W2, b2):
    grid = (N // BN,)
    return pl.pallas_call(
        _mlp_body,
        grid=grid,
        in_specs=[
            pl.BlockSpec((NC, BN, D), lambda i: (0, i, 0)),
            pl.BlockSpec((BN, D), lambda i: (i, 0)),
            pl.BlockSpec((1, 1), lambda i: (0, 0)),
            pl.BlockSpec((D, D), lambda i: (0, 0)),
            pl.BlockSpec((1, D), lambda i: (0, 0)),
            pl.BlockSpec((D, D), lambda i: (0, 0)),
            pl.BlockSpec((1, D), lambda i: (0, 0)),
        ],
        out_specs=pl.BlockSpec((BN, D), lambda i: (i, 0)),
        out_shape=jax.ShapeDtypeStruct((N, D), jnp.float32),
    )(partials, x, scale, W1, b1, W2, b2)


@jax.jit
def kernel(x, edge_index, W1, b1, W2, b2, eps):
    src = edge_index[0].reshape(NW, R, K)
    dst = edge_index[1].reshape(NW, R, K)
    partials = _sc_agg(src, dst, x)
    scale = (1.0 + eps[0]).reshape(1, 1)
    return _mlp(partials, x, scale, W1, b1.reshape(1, D), W2, b2.reshape(1, D))


# speedup vs baseline: 9.9834x; 1.2046x over previous
import functools

import jax
import jax.numpy as jnp
from jax import lax
from jax.experimental import pallas as pl
from jax.experimental.pallas import tpu as pltpu
from jax.experimental.pallas import tpu_sc as plsc

NC = 2
NS = 16
NW = NC * NS

N = 10000
E = 320000
D = 128

T = E // NW
K = 100
R = T // K
HALVES = 2
HR = R // HALVES
NP = 10240
ROWS_PER_TILE = NP // NS
ZR = 64
assert ROWS_PER_TILE % ZR == 0 and T == R * K and NP % NS == 0 and ZR <= K
assert R % HALVES == 0 and HR % 2 == 0


def _sc_agg_body(src_hbm, dst_hbm, x_hbm, out_hbm,
                 src_v, dst_v, rows_a, rows_b, acc,
                 gsem_a, gsem_b, ssem_a, ssem_b):
    c = lax.axis_index("c")
    s = lax.axis_index("s")
    wid = s * NC + c

    zeros16 = jnp.zeros((16,), jnp.float32)

    def zrow(i, carry):
        for cc in range(D // 16):
            rows_a[i, pl.ds(cc * 16, 16)] = zeros16
        return carry

    lax.fori_loop(0, ZR, zrow, 0)

    def zcp(i, carry):
        pltpu.sync_copy(rows_a.at[pl.ds(0, ZR)],
                        acc.at[pl.ds(s * ROWS_PER_TILE + i * ZR, ZR)])
        return carry

    lax.fori_loop(0, ROWS_PER_TILE // ZR, zcp, 0)
    plsc.subcore_barrier()

    def gather(j, buf, sem):
        return pltpu.async_copy(x_hbm.at[src_v.at[j]], buf, sem)

    def gather_wait(buf, sem):
        pltpu.make_async_copy(x_hbm.at[src_v.at[0]], buf, sem).wait()

    def scat(j, buf, sem):
        return pltpu.async_copy(buf, acc.at[dst_v.at[j]], sem, add=True)

    def scat_wait(buf, sem):
        pltpu.make_async_copy(buf, acc.at[dst_v.at[0]], sem).wait()

    for h in range(HALVES):
        pltpu.sync_copy(src_hbm.at[wid, h], src_v)
        pltpu.sync_copy(dst_hbm.at[wid, h], dst_v)
        gather(0, rows_a, gsem_a)
        gather(1, rows_b, gsem_b)

        def pair(i, carry):
            j = 2 * i
            gather_wait(rows_a, gsem_a)
            scat(j, rows_a, ssem_a)
            gather_wait(rows_b, gsem_b)
            scat(j + 1, rows_b, ssem_b)

            @pl.when(j + 2 < HR)
            def _():
                scat_wait(rows_a, ssem_a)
                gather(j + 2, rows_a, gsem_a)
                scat_wait(rows_b, ssem_b)
                gather(j + 3, rows_b, gsem_b)

            return carry

        lax.fori_loop(0, HR // 2, pair, 0)
        scat_wait(rows_a, ssem_a)
        scat_wait(rows_b, ssem_b)
    plsc.subcore_barrier()

    pltpu.sync_copy(
        acc.at[pl.ds(s * ROWS_PER_TILE, ROWS_PER_TILE)],
        out_hbm.at[c, pl.ds(s * ROWS_PER_TILE, ROWS_PER_TILE)],
    )


_sc_agg = functools.partial(
    pl.kernel,
    out_type=jax.ShapeDtypeStruct((NC, NP, D), jnp.float32),
    mesh=plsc.VectorSubcoreMesh(core_axis_name="c", subcore_axis_name="s"),
    scratch_types=[
        pltpu.VMEM((HR, K), jnp.int32),
        pltpu.VMEM((HR, K), jnp.int32),
        pltpu.VMEM((K, D), jnp.float32),
        pltpu.VMEM((K, D), jnp.float32),
        pltpu.VMEM_SHARED((NP, D), jnp.float32),
        pltpu.SemaphoreType.DMA,
        pltpu.SemaphoreType.DMA,
        pltpu.SemaphoreType.DMA,
        pltpu.SemaphoreType.DMA,
    ],
)(_sc_agg_body)


BN = 1000


def _mlp_body(p_ref, x_ref, sc_ref, w1_ref, b1_ref, w2_ref, b2_ref, o_ref):
    y = p_ref[0] + p_ref[1] + sc_ref[0, 0] * x_ref[...]
    h = jnp.dot(y, w1_ref[...], preferred_element_type=jnp.float32)
    h = jnp.maximum(h + b1_ref[...], 0.0)
    o = jnp.dot(h, w2_ref[...], preferred_element_type=jnp.float32)
    o_ref[...] = o + b2_ref[...]


def _mlp(partials, x, scale, W1, b1, W2, b2):
    grid = (N // BN,)
    return pl.pallas_call(
        _mlp_body,
        grid=grid,
        in_specs=[
            pl.BlockSpec((NC, BN, D), lambda i: (0, i, 0)),
            pl.BlockSpec((BN, D), lambda i: (i, 0)),
            pl.BlockSpec((1, 1), lambda i: (0, 0)),
            pl.BlockSpec((D, D), lambda i: (0, 0)),
            pl.BlockSpec((1, D), lambda i: (0, 0)),
            pl.BlockSpec((D, D), lambda i: (0, 0)),
            pl.BlockSpec((1, D), lambda i: (0, 0)),
        ],
        out_specs=pl.BlockSpec((BN, D), lambda i: (i, 0)),
        out_shape=jax.ShapeDtypeStruct((N, D), jnp.float32),
    )(partials, x, scale, W1, b1, W2, b2)


@jax.jit
def kernel(x, edge_index, W1, b1, W2, b2, eps):
    src = edge_index[0].reshape(NW, HALVES, HR, K)
    dst = edge_index[1].reshape(NW, HALVES, HR, K)
    partials = _sc_agg(src, dst, x)
    scale = (1.0 + eps[0]).reshape(1, 1)
    return _mlp(partials, x, scale, W1, b1.reshape(1, D), W2, b2.reshape(1, D))


# speedup vs baseline: 10.9872x; 1.1006x over previous
import functools

import jax
import jax.numpy as jnp
from jax import lax
from jax.experimental import pallas as pl
from jax.experimental.pallas import tpu as pltpu
from jax.experimental.pallas import tpu_sc as plsc

NC = 2
NS = 16
NW = NC * NS

N = 10000
E = 320000
D = 128

T = E // NW
K = 40
R = T // K
NB = 4
HALVES = 5
HR = R // HALVES
NBLK = -(-HR // NB)
NP = 10240
ROWS_PER_TILE = NP // NS
ZR = 40
assert T == R * K and NP % NS == 0 and ZR <= K and ROWS_PER_TILE % ZR == 0
assert R % HALVES == 0 and HR >= NB and ROWS_PER_TILE % 8 == 0


def _sc_agg_body(src_hbm, dst_hbm, x_hbm, out_hbm,
                 src_v, dst_v, *scr):
    bufs = scr[:NB]
    acc = scr[NB]
    gsems = scr[NB + 1:2 * NB + 1]
    ssems = scr[2 * NB + 1:]
    c = lax.axis_index("c")
    s = lax.axis_index("s")
    wid = s * NC + c

    zeros16 = jnp.zeros((16,), jnp.float32)

    def zrow(i, carry):
        for cc in range(D // 16):
            bufs[0][i, pl.ds(cc * 16, 16)] = zeros16
        return carry

    lax.fori_loop(0, ZR, zrow, 0)

    def zcp(i, carry):
        pltpu.sync_copy(bufs[0].at[pl.ds(0, ZR)],
                        acc.at[pl.ds(s * ROWS_PER_TILE + i * ZR, ZR)])
        return carry

    lax.fori_loop(0, ROWS_PER_TILE // ZR, zcp, 0)
    plsc.subcore_barrier()

    def gather(j, buf, sem):
        return pltpu.async_copy(x_hbm.at[src_v.at[j]], buf, sem)

    def gather_wait(buf, sem):
        pltpu.make_async_copy(x_hbm.at[src_v.at[0]], buf, sem).wait()

    def scat(j, buf, sem):
        return pltpu.async_copy(buf, acc.at[dst_v.at[j]], sem, add=True)

    def scat_wait(buf, sem):
        pltpu.make_async_copy(buf, acc.at[dst_v.at[0]], sem).wait()

    for h in range(HALVES):
        pltpu.sync_copy(src_hbm.at[wid, h], src_v)
        pltpu.sync_copy(dst_hbm.at[wid, h], dst_v)
        for b in range(NB):
            gather(b, bufs[b], gsems[b])

        def block(i, carry):
            base = NB * i
            for b in range(NB):
                j = base + b

                @pl.when(j < HR)
                def _(b=b, j=j):
                    gather_wait(bufs[b], gsems[b])
                    scat(j, bufs[b], ssems[b])

            for b in range(NB):
                nxt = base + NB + b

                @pl.when(nxt < HR)
                def _(b=b, nxt=nxt):
                    scat_wait(bufs[b], ssems[b])
                    gather(nxt, bufs[b], gsems[b])

            return carry

        lax.fori_loop(0, NBLK, block, 0)
        for b in range(NB):
            scat_wait(bufs[b], ssems[b])
    plsc.subcore_barrier()

    pltpu.sync_copy(
        acc.at[pl.ds(s * ROWS_PER_TILE, ROWS_PER_TILE)],
        out_hbm.at[c, pl.ds(s * ROWS_PER_TILE, ROWS_PER_TILE)],
    )


_sc_agg = functools.partial(
    pl.kernel,
    out_type=jax.ShapeDtypeStruct((NC, NP, D), jnp.float32),
    mesh=plsc.VectorSubcoreMesh(core_axis_name="c", subcore_axis_name="s"),
    scratch_types=(
        [pltpu.VMEM((HR, K), jnp.int32),
         pltpu.VMEM((HR, K), jnp.int32)]
        + [pltpu.VMEM((K, D), jnp.float32)] * NB
        + [pltpu.VMEM_SHARED((NP, D), jnp.float32)]
        + [pltpu.SemaphoreType.DMA] * (2 * NB)
    ),
)(_sc_agg_body)


BN = 1000


def _mlp_body(p_ref, x_ref, sc_ref, w1_ref, b1_ref, w2_ref, b2_ref, o_ref):
    y = p_ref[0] + p_ref[1] + sc_ref[0, 0] * x_ref[...]
    h = jnp.dot(y, w1_ref[...], preferred_element_type=jnp.float32)
    h = jnp.maximum(h + b1_ref[...], 0.0)
    o = jnp.dot(h, w2_ref[...], preferred_element_type=jnp.float32)
    o_ref[...] = o + b2_ref[...]


def _mlp(partials, x, scale, W1, b1, W2, b2):
    grid = (N // BN,)
    return pl.pallas_call(
        _mlp_body,
        grid=grid,
        in_specs=[
            pl.BlockSpec((NC, BN, D), lambda i: (0, i, 0)),
            pl.BlockSpec((BN, D), lambda i: (i, 0)),
            pl.BlockSpec((1, 1), lambda i: (0, 0)),
            pl.BlockSpec((D, D), lambda i: (0, 0)),
            pl.BlockSpec((1, D), lambda i: (0, 0)),
            pl.BlockSpec((D, D), lambda i: (0, 0)),
            pl.BlockSpec((1, D), lambda i: (0, 0)),
        ],
        out_specs=pl.BlockSpec((BN, D), lambda i: (i, 0)),
        out_shape=jax.ShapeDtypeStruct((N, D), jnp.float32),
    )(partials, x, scale, W1, b1, W2, b2)


@jax.jit
def kernel(x, edge_index, W1, b1, W2, b2, eps):
    src = edge_index[0].reshape(NW, HALVES, HR, K)
    dst = edge_index[1].reshape(NW, HALVES, HR, K)
    partials = _sc_agg(src, dst, x)
    scale = (1.0 + eps[0]).reshape(1, 1)
    return _mlp(partials, x, scale, W1, b1.reshape(1, D), W2, b2.reshape(1, D))


# speedup vs baseline: 11.0847x; 1.0089x over previous
import functools

import jax
import jax.numpy as jnp
from jax import lax
from jax.experimental import pallas as pl
from jax.experimental.pallas import tpu as pltpu
from jax.experimental.pallas import tpu_sc as plsc

NC = 2
NS = 16
NW = NC * NS

N = 10000
E = 320000
D = 128

T = E // NW
K = 50
R = T // K
NB = 4
HALVES = 5
HR = R // HALVES
NBLK = -(-HR // NB)
NP = 10240
ROWS_PER_TILE = NP // NS
ZR = 40
assert T == R * K and NP % NS == 0 and ZR <= K and ROWS_PER_TILE % ZR == 0
assert R % HALVES == 0 and HR >= NB and ROWS_PER_TILE % 8 == 0


def _sc_agg_body(src_hbm, dst_hbm, x_hbm, out_hbm,
                 src_v, dst_v, *scr):
    bufs = scr[:NB]
    acc = scr[NB]
    gsems = scr[NB + 1:2 * NB + 1]
    ssems = scr[2 * NB + 1:]
    c = lax.axis_index("c")
    s = lax.axis_index("s")
    wid = s * NC + c

    ring = bufs[1:] + (bufs[0],)
    gring = gsems[1:] + (gsems[0],)
    sring = ssems[1:] + (ssems[0],)

    def gather(j, buf, sem):
        return pltpu.async_copy(x_hbm.at[src_v.at[j]], buf, sem)

    def gather_wait(buf, sem):
        pltpu.make_async_copy(x_hbm.at[src_v.at[0]], buf, sem).wait()

    def scat(j, buf, sem):
        return pltpu.async_copy(buf, acc.at[dst_v.at[j]], sem, add=True)

    def scat_wait(buf, sem):
        pltpu.make_async_copy(buf, acc.at[dst_v.at[0]], sem).wait()

    pltpu.sync_copy(src_hbm.at[wid, 0], src_v)
    pltpu.sync_copy(dst_hbm.at[wid, 0], dst_v)
    for b in range(NB - 1):
        gather(b, ring[b], gring[b])

    zeros16 = jnp.zeros((16,), jnp.float32)

    def zrow(i, carry):
        for cc in range(D // 16):
            bufs[0][i, pl.ds(cc * 16, 16)] = zeros16
        return carry

    lax.fori_loop(0, ZR, zrow, 0)

    def zcp(i, carry):
        pltpu.sync_copy(bufs[0].at[pl.ds(0, ZR)],
                        acc.at[pl.ds(s * ROWS_PER_TILE + i * ZR, ZR)])
        return carry

    lax.fori_loop(0, ROWS_PER_TILE // ZR, zcp, 0)
    gather(NB - 1, ring[NB - 1], gring[NB - 1])
    plsc.subcore_barrier()

    for h in range(HALVES):
        if h > 0:
            pltpu.sync_copy(src_hbm.at[wid, h], src_v)
            pltpu.sync_copy(dst_hbm.at[wid, h], dst_v)
            for b in range(NB):
                gather(b, ring[b], gring[b])

        def block(i, carry):
            base = NB * i
            for b in range(NB):
                j = base + b

                @pl.when(j < HR)
                def _(b=b, j=j):
                    gather_wait(ring[b], gring[b])
                    scat(j, ring[b], sring[b])

            for b in range(NB):
                nxt = base + NB + b

                @pl.when(nxt < HR)
                def _(b=b, nxt=nxt):
                    scat_wait(ring[b], sring[b])
                    gather(nxt, ring[b], gring[b])

            return carry

        lax.fori_loop(0, NBLK, block, 0)
        for b in range(NB):
            scat_wait(ring[b], sring[b])
    plsc.subcore_barrier()

    pltpu.sync_copy(
        acc.at[pl.ds(s * ROWS_PER_TILE, ROWS_PER_TILE)],
        out_hbm.at[c, pl.ds(s * ROWS_PER_TILE, ROWS_PER_TILE)],
    )


_sc_agg = functools.partial(
    pl.kernel,
    out_type=jax.ShapeDtypeStruct((NC, NP, D), jnp.float32),
    mesh=plsc.VectorSubcoreMesh(core_axis_name="c", subcore_axis_name="s"),
    scratch_types=(
        [pltpu.VMEM((HR, K), jnp.int32),
         pltpu.VMEM((HR, K), jnp.int32)]
        + [pltpu.VMEM((K, D), jnp.float32)] * NB
        + [pltpu.VMEM_SHARED((NP, D), jnp.float32)]
        + [pltpu.SemaphoreType.DMA] * (2 * NB)
    ),
)(_sc_agg_body)


BN = 1000


def _mlp_body(p_ref, x_ref, sc_ref, w1_ref, b1_ref, w2_ref, b2_ref, o_ref):
    y = p_ref[0] + p_ref[1] + sc_ref[0, 0] * x_ref[...]
    h = jnp.dot(y, w1_ref[...], preferred_element_type=jnp.float32)
    h = jnp.maximum(h + b1_ref[...], 0.0)
    o = jnp.dot(h, w2_ref[...], preferred_element_type=jnp.float32)
    o_ref[...] = o + b2_ref[...]


def _mlp(partials, x, scale, W1, b1, W2, b2):
    grid = (N // BN,)
    return pl.pallas_call(
        _mlp_body,
        grid=grid,
        in_specs=[
            pl.BlockSpec((NC, BN, D), lambda i: (0, i, 0)),
            pl.BlockSpec((BN, D), lambda i: (i, 0)),
            pl.BlockSpec((1, 1), lambda i: (0, 0)),
            pl.BlockSpec((D, D), lambda i: (0, 0)),
            pl.BlockSpec((1, D), lambda i: (0, 0)),
            pl.BlockSpec((D, D), lambda i: (0, 0)),
            pl.BlockSpec((1, D), lambda i: (0, 0)),
        ],
        out_specs=pl.BlockSpec((BN, D), lambda i: (i, 0)),
        out_shape=jax.ShapeDtypeStruct((N, D), jnp.float32),
    )(partials, x, scale, W1, b1, W2, b2)


@jax.jit
def kernel(x, edge_index, W1, b1, W2, b2, eps):
    src = edge_index[0].reshape(NW, HALVES, HR, K)
    dst = edge_index[1].reshape(NW, HALVES, HR, K)
    partials = _sc_agg(src, dst, x)
    scale = (1.0 + eps[0]).reshape(1, 1)
    return _mlp(partials, x, scale, W1, b1.reshape(1, D), W2, b2.reshape(1, D))


# speedup vs baseline: 11.5707x; 1.0438x over previous
import functools

import jax
import jax.numpy as jnp
from jax import lax
from jax.experimental import pallas as pl
from jax.experimental.pallas import tpu as pltpu
from jax.experimental.pallas import tpu_sc as plsc

NC = 2
NS = 16
NW = NC * NS

N = 10000
E = 320000
D = 128

T = E // NW
K = 40
R = T // K
NB = 6
HALVES = 5
HR = R // HALVES
NBLK = -(-HR // NB)
NP = 10240
ROWS_PER_TILE = NP // NS
ZR = 40
assert T == R * K and NP % NS == 0 and ZR <= K and ROWS_PER_TILE % ZR == 0
assert R % HALVES == 0 and HR >= NB and ROWS_PER_TILE % 8 == 0


def _sc_agg_body(src_hbm, dst_hbm, x_hbm, out_hbm,
                 src_v, dst_v, *scr):
    bufs = scr[:NB]
    acc = scr[NB]
    gsems = scr[NB + 1:2 * NB + 1]
    ssems = scr[2 * NB + 1:]
    c = lax.axis_index("c")
    s = lax.axis_index("s")
    wid = s * NC + c

    ring = bufs[1:] + (bufs[0],)
    gring = gsems[1:] + (gsems[0],)
    sring = ssems[1:] + (ssems[0],)

    def gather(j, buf, sem):
        return pltpu.async_copy(x_hbm.at[src_v.at[j]], buf, sem)

    def gather_wait(buf, sem):
        pltpu.make_async_copy(x_hbm.at[src_v.at[0]], buf, sem).wait()

    def scat(j, buf, sem):
        return pltpu.async_copy(buf, acc.at[dst_v.at[j]], sem, add=True)

    def scat_wait(buf, sem):
        pltpu.make_async_copy(buf, acc.at[dst_v.at[0]], sem).wait()

    pltpu.sync_copy(src_hbm.at[wid, 0], src_v)
    pltpu.sync_copy(dst_hbm.at[wid, 0], dst_v)
    for b in range(NB - 1):
        gather(b, ring[b], gring[b])

    zeros16 = jnp.zeros((16,), jnp.float32)

    def zrow(i, carry):
        for cc in range(D // 16):
            bufs[0][i, pl.ds(cc * 16, 16)] = zeros16
        return carry

    lax.fori_loop(0, ZR, zrow, 0)

    def zcp(i, carry):
        pltpu.sync_copy(bufs[0].at[pl.ds(0, ZR)],
                        acc.at[pl.ds(s * ROWS_PER_TILE + i * ZR, ZR)])
        return carry

    lax.fori_loop(0, ROWS_PER_TILE // ZR, zcp, 0)
    gather(NB - 1, ring[NB - 1], gring[NB - 1])
    plsc.subcore_barrier()

    for h in range(HALVES):
        if h > 0:
            pltpu.sync_copy(src_hbm.at[wid, h], src_v)
            pltpu.sync_copy(dst_hbm.at[wid, h], dst_v)
            for b in range(NB):
                gather(b, ring[b], gring[b])

        def block(i, carry):
            base = NB * i
            for b in range(NB):
                j = base + b

                @pl.when(j < HR)
                def _(b=b, j=j):
                    gather_wait(ring[b], gring[b])
                    scat(j, ring[b], sring[b])

            for b in range(NB):
                nxt = base + NB + b

                @pl.when(nxt < HR)
                def _(b=b, nxt=nxt):
                    scat_wait(ring[b], sring[b])
                    gather(nxt, ring[b], gring[b])

            return carry

        lax.fori_loop(0, NBLK, block, 0)
        for b in range(NB):
            scat_wait(ring[b], sring[b])
    plsc.subcore_barrier()

    pltpu.sync_copy(
        acc.at[pl.ds(s * ROWS_PER_TILE, ROWS_PER_TILE)],
        out_hbm.at[c, pl.ds(s * ROWS_PER_TILE, ROWS_PER_TILE)],
    )


_sc_agg = functools.partial(
    pl.kernel,
    out_type=jax.ShapeDtypeStruct((NC, NP, D), jnp.float32),
    mesh=plsc.VectorSubcoreMesh(core_axis_name="c", subcore_axis_name="s"),
    scratch_types=(
        [pltpu.VMEM((HR, K), jnp.int32),
         pltpu.VMEM((HR, K), jnp.int32)]
        + [pltpu.VMEM((K, D), jnp.float32)] * NB
        + [pltpu.VMEM_SHARED((NP, D), jnp.float32)]
        + [pltpu.SemaphoreType.DMA] * (2 * NB)
    ),
)(_sc_agg_body)


BN = 1000


def _mlp_body(p_ref, x_ref, sc_ref, w1_ref, b1_ref, w2_ref, b2_ref, o_ref):
    y = p_ref[0] + p_ref[1] + sc_ref[0, 0] * x_ref[...]
    h = jnp.dot(y, w1_ref[...], preferred_element_type=jnp.float32)
    h = jnp.maximum(h + b1_ref[...], 0.0)
    o = jnp.dot(h, w2_ref[...], preferred_element_type=jnp.float32)
    o_ref[...] = o + b2_ref[...]


def _mlp(partials, x, scale, W1, b1, W2, b2):
    grid = (N // BN,)
    return pl.pallas_call(
        _mlp_body,
        grid=grid,
        in_specs=[
            pl.BlockSpec((NC, BN, D), lambda i: (0, i, 0)),
            pl.BlockSpec((BN, D), lambda i: (i, 0)),
            pl.BlockSpec((1, 1), lambda i: (0, 0)),
            pl.BlockSpec((D, D), lambda i: (0, 0)),
            pl.BlockSpec((1, D), lambda i: (0, 0)),
            pl.BlockSpec((D, D), lambda i: (0, 0)),
            pl.BlockSpec((1, D), lambda i: (0, 0)),
        ],
        out_specs=pl.BlockSpec((BN, D), lambda i: (i, 0)),
        out_shape=jax.ShapeDtypeStruct((N, D), jnp.float32),
    )(partials, x, scale, W1, b1, W2, b2)


@jax.jit
def kernel(x, edge_index, W1, b1, W2, b2, eps):
    src = edge_index[0].reshape(NW, HALVES, HR, K)
    dst = edge_index[1].reshape(NW, HALVES, HR, K)
    partials = _sc_agg(src, dst, x)
    scale = (1.0 + eps[0]).reshape(1, 1)
    return _mlp(partials, x, scale, W1, b1.reshape(1, D), W2, b2.reshape(1, D))
